# baseline (speedup 1.0000x reference)
import jax
import jax.numpy as jnp
from jax import lax
from jax.experimental import pallas as pl

_B, _C, _T = 16, 64, 1000
_K = 64
_KL = 9
_TV = _T - _KL + 1
_R = 8
_NB = 4
_DO = 64
_F = 2 * _K

_RB = 8


def _feats_kernel(x_ref, kern_ref, bias_ref, out_ref):
    xw = x_ref[:, :]
    shifts = [xw[:, j:j + _TV] for j in range(_KL)]
    cols = []
    mx_cols = []
    for k in range(_K):
        acc = shifts[0] * kern_ref[k, 0]
        for j in range(1, _KL):
            acc = acc + shifts[j] * kern_ref[k, j]
        bk = bias_ref[0, k]
        ind = (acc > -bk).astype(jnp.float32)
        ppv = jnp.sum(ind, axis=1) * (1.0 / _TV)
        mxv = jnp.max(acc, axis=1) + bk
        cols.append(ppv.reshape(_RB, 1))
        mx_cols.append(mxv.reshape(_RB, 1))
    out_ref[:, :] = jnp.concatenate(cols + mx_cols, axis=1)


def _pool_head_kernel(rid_ref, feats_ref, wc_ref, bc_ref, out_ref):
    rid = rid_ref[:, :]
    rows = lax.broadcasted_iota(jnp.int32, (_R, _C), 0)
    m = (rid == rows).astype(jnp.float32)
    counts = jnp.maximum(jnp.sum(m, axis=1, keepdims=True), 1.0)
    mn = m / counts
    pooled = jnp.dot(mn, feats_ref[:, :], preferred_element_type=jnp.float32)
    for b in range(_B):
        pb = pooled[:, b * _F:(b + 1) * _F]
        ob = jnp.dot(pb, wc_ref[:, :], preferred_element_type=jnp.float32)
        out_ref[b * _R:(b + 1) * _R, :] = ob + bc_ref[:, :]


def kernel(x, region_ids, kernels, biases, W, b):
    xr = x.reshape(_B * _C, _T)
    k2 = kernels.reshape(_K, _KL).astype(jnp.float32)
    b2 = biases.reshape(1, _K).astype(jnp.float32)

    feats = pl.pallas_call(
        _feats_kernel,
        grid=(_B * _C // _RB,),
        in_specs=[
            pl.BlockSpec((_RB, _T), lambda i: (i, 0)),
            pl.BlockSpec((_K, _KL), lambda i: (0, 0)),
            pl.BlockSpec((1, _K), lambda i: (0, 0)),
        ],
        out_specs=pl.BlockSpec((_RB, _F), lambda i: (i, 0)),
        out_shape=jax.ShapeDtypeStruct((_B * _C, _F), jnp.float32),
    )(xr, k2, b2)

    feats_t = feats.reshape(_B, _C, _F).transpose(1, 0, 2).reshape(_C, _B * _F)
    wc = W.transpose(1, 0, 2).reshape(_F, _NB * _DO)
    bc = b.reshape(1, _NB * _DO)
    rid = region_ids.astype(jnp.int32).reshape(1, _C)

    out = pl.pallas_call(
        _pool_head_kernel,
        in_specs=[
            pl.BlockSpec((1, _C), lambda: (0, 0)),
            pl.BlockSpec((_C, _B * _F), lambda: (0, 0)),
            pl.BlockSpec((_F, _NB * _DO), lambda: (0, 0)),
            pl.BlockSpec((1, _NB * _DO), lambda: (0, 0)),
        ],
        out_specs=pl.BlockSpec((_B * _R, _NB * _DO), lambda: (0, 0)),
        out_shape=jax.ShapeDtypeStruct((_B * _R, _NB * _DO), jnp.float32),
    )(rid, feats_t, wc, bc)

    out = out.reshape(_B, _R, _NB, _DO).transpose(0, 2, 1, 3)
    return out.reshape(_B, _NB, _R * _DO)


# speedup vs baseline: 38.1455x; 38.1455x over previous
import jax
import jax.numpy as jnp
from jax import lax
from jax.experimental import pallas as pl

_B, _C, _T = 16, 64, 1000
_K = 64
_KL = 9
_TV = _T - _KL + 1
_R = 8
_NB = 4
_DO = 64
_F = 2 * _K

_L = 128
_NCH = 8
_TP = _NCH * _L + (_KL - 1)
_G = 32


def _feats_kernel(x_ref, m1_ref, m2_ref, thr_ref, mask_ref, tones_ref,
                  bias_ref, out_ref):
    a1 = jnp.concatenate(
        [x_ref[:, ch * _L:ch * _L + _L] for ch in range(_NCH)], axis=0)
    a2 = jnp.concatenate(
        [x_ref[:, ch * _L + _L:ch * _L + _L + (_KL - 1)]
         for ch in range(_NCH)], axis=0)
    conv = (jnp.dot(a1, m1_ref[:, :], preferred_element_type=jnp.float32)
            + jnp.dot(a2, m2_ref[:, :], preferred_element_type=jnp.float32)
            ).astype(jnp.bfloat16)
    thr = thr_ref[:, :]
    nrows = (_NCH - 1) * _G
    conv_main = conv[:nrows, :]
    conv_last = conv[nrows:, :]
    mb = mask_ref[:, :] > jnp.bfloat16(0.0)
    ind_main = (conv_main > thr).astype(jnp.bfloat16)
    ind_last = ((conv_last > thr) & mb).astype(jnp.bfloat16)
    ind = jnp.concatenate([ind_main, ind_last], axis=0)
    counts = jnp.dot(ind, tones_ref[:, :], preferred_element_type=jnp.float32)
    total = counts[:_G, :]
    for ch in range(1, _NCH):
        total = total + counts[ch * _G:(ch + 1) * _G, :]
    ppv = total * (1.0 / _TV)
    neg = jnp.full(conv_last.shape, jnp.bfloat16(-3e38))
    conv_last_m = jnp.where(mb, conv_last, neg)
    m = conv_main[:_G, :]
    for ch in range(1, _NCH - 1):
        m = jnp.maximum(m, conv_main[ch * _G:(ch + 1) * _G, :])
    m = jnp.maximum(m, conv_last_m)
    m3 = m.reshape(_G, _K, _L)
    mx = jnp.max(m3, axis=-1).astype(jnp.float32) + bias_ref[:, :]
    out_ref[:, :] = jnp.concatenate([ppv, mx], axis=1)


def _pool_head_kernel(rid_ref, feats_ref, wc_ref, bc_ref, out_ref):
    rid = rid_ref[:, :]
    rows = lax.broadcasted_iota(jnp.int32, (_R, _C), 0)
    m = (rid == rows).astype(jnp.float32)
    counts = jnp.maximum(jnp.sum(m, axis=1, keepdims=True), 1.0)
    mn = m / counts
    pooled = jnp.dot(mn, feats_ref[:, :], preferred_element_type=jnp.float32)
    for b in range(_B):
        pb = pooled[:, b * _F:(b + 1) * _F]
        ob = jnp.dot(pb, wc_ref[:, :], preferred_element_type=jnp.float32)
        out_ref[b * _R:(b + 1) * _R, :] = ob + bc_ref[:, :]


def kernel(x, region_ids, kernels, biases, W, b):
    xr = x.reshape(_B * _C, _T)
    xp = jnp.pad(xr, ((0, 0), (0, _TP - _T))).astype(jnp.bfloat16)
    k2 = kernels.reshape(_K, _KL).astype(jnp.float32)

    s1 = jnp.arange(_L)[:, None]
    s2 = jnp.arange(_KL - 1)[:, None] + _L
    t = jnp.arange(_L)[None, :]
    d1 = s1 - t
    d2 = s2 - t
    km1 = jnp.where((d1 >= 0) & (d1 < _KL),
                    k2[:, jnp.clip(d1, 0, _KL - 1)], 0.0)
    km2 = jnp.where((d2 >= 0) & (d2 < _KL),
                    k2[:, jnp.clip(d2, 0, _KL - 1)], 0.0)
    m1 = km1.transpose(1, 0, 2).reshape(_L, _K * _L).astype(jnp.bfloat16)
    m2 = km2.transpose(1, 0, 2).reshape(_KL - 1, _K * _L).astype(jnp.bfloat16)

    thr = jnp.repeat(-biases.astype(jnp.float32), _L).reshape(1, _K * _L)
    thr = thr.astype(jnp.bfloat16)
    lane_t = jnp.arange(_K * _L) % _L
    maskv = (lane_t < (_TV - (_NCH - 1) * _L)).astype(jnp.bfloat16)
    maskv = maskv.reshape(1, _K * _L)
    tones = (jnp.arange(_K * _L)[:, None] // _L
             == jnp.arange(_K)[None, :]).astype(jnp.bfloat16)
    b2 = biases.reshape(1, _K).astype(jnp.float32)

    feats = pl.pallas_call(
        _feats_kernel,
        grid=(_B * _C // _G,),
        in_specs=[
            pl.BlockSpec((_G, _TP), lambda i: (i, 0)),
            pl.BlockSpec((_L, _K * _L), lambda i: (0, 0)),
            pl.BlockSpec((_KL - 1, _K * _L), lambda i: (0, 0)),
            pl.BlockSpec((1, _K * _L), lambda i: (0, 0)),
            pl.BlockSpec((1, _K * _L), lambda i: (0, 0)),
            pl.BlockSpec((_K * _L, _K), lambda i: (0, 0)),
            pl.BlockSpec((1, _K), lambda i: (0, 0)),
        ],
        out_specs=pl.BlockSpec((_G, _F), lambda i: (i, 0)),
        out_shape=jax.ShapeDtypeStruct((_B * _C, _F), jnp.float32),
    )(xp, m1, m2, thr, maskv, tones, b2)

    feats_t = feats.reshape(_B, _C, _F).transpose(1, 0, 2).reshape(_C, _B * _F)
    wc = W.transpose(1, 0, 2).reshape(_F, _NB * _DO)
    bc = b.reshape(1, _NB * _DO)
    rid = region_ids.astype(jnp.int32).reshape(1, _C)

    out = pl.pallas_call(
        _pool_head_kernel,
        in_specs=[
            pl.BlockSpec((1, _C), lambda: (0, 0)),
            pl.BlockSpec((_C, _B * _F), lambda: (0, 0)),
            pl.BlockSpec((_F, _NB * _DO), lambda: (0, 0)),
            pl.BlockSpec((1, _NB * _DO), lambda: (0, 0)),
        ],
        out_specs=pl.BlockSpec((_B * _R, _NB * _DO), lambda: (0, 0)),
        out_shape=jax.ShapeDtypeStruct((_B * _R, _NB * _DO), jnp.float32),
    )(rid, feats_t, wc, bc)

    out = out.reshape(_B, _R, _NB, _DO).transpose(0, 2, 1, 3)
    return out.reshape(_B, _NB, _R * _DO)


# speedup vs baseline: 67.3542x; 1.7657x over previous
import jax
import jax.numpy as jnp
from jax import lax
from jax.experimental import pallas as pl
from jax.experimental.pallas import tpu as pltpu

_B, _C, _T = 16, 64, 1000
_K = 64
_KL = 9
_TV = _T - _KL + 1
_R = 8
_NB = 4
_DO = 64
_F = 2 * _K

_L = 128
_NCH = 8
_W = _L + _KL - 1
_TP = _NCH * _L + (_KL - 1)
_G = 32


def _feats_kernel(x_ref, kern_ref, thr_ref, mask_ref, tones_ref, bias_ref,
                  out_ref, m_ref):
    @pl.when(pl.program_id(0) == 0)
    def _build_m():
        d = (lax.broadcasted_iota(jnp.int32, (_W, _L), 0)
             - lax.broadcasted_iota(jnp.int32, (_W, _L), 1))
        diags = [(d == j).astype(jnp.float32) for j in range(_KL)]
        for k in range(_K):
            tb = diags[0] * kern_ref[k, 0]
            for j in range(1, _KL):
                tb = tb + diags[j] * kern_ref[k, j]
            m_ref[:, k * _L:(k + 1) * _L] = tb.astype(jnp.bfloat16)

    a = jnp.concatenate(
        [x_ref[:, ch * _L:ch * _L + _W] for ch in range(_NCH)], axis=0)
    conv = jnp.dot(a, m_ref[:, :],
                   preferred_element_type=jnp.float32).astype(jnp.bfloat16)
    thr = thr_ref[:, :]
    nrows = (_NCH - 1) * _G
    conv_main = conv[:nrows, :]
    conv_last = conv[nrows:, :]
    mb = mask_ref[:, :] > jnp.bfloat16(0.0)
    ind_main = (conv_main > thr).astype(jnp.bfloat16)
    ind_last = ((conv_last > thr) & mb).astype(jnp.bfloat16)
    ind = jnp.concatenate([ind_main, ind_last], axis=0)
    counts = jnp.dot(ind, tones_ref[:, :], preferred_element_type=jnp.float32)
    total = counts[:_G, :]
    for ch in range(1, _NCH):
        total = total + counts[ch * _G:(ch + 1) * _G, :]
    ppv = total * (1.0 / _TV)
    neg = jnp.full(conv_last.shape, jnp.bfloat16(-3e38))
    conv_last_m = jnp.where(mb, conv_last, neg)
    m = conv_main[:_G, :]
    for ch in range(1, _NCH - 1):
        m = jnp.maximum(m, conv_main[ch * _G:(ch + 1) * _G, :])
    m = jnp.maximum(m, conv_last_m)
    m3 = m.reshape(_G, _K, _L)
    mx = jnp.max(m3, axis=-1).astype(jnp.float32) + bias_ref[:, :]
    out_ref[:, :] = jnp.concatenate([ppv, mx], axis=1)


def _pool_head_kernel(rid_ref, feats_ref, wc_ref, bc_ref, out_ref):
    rid = rid_ref[:, :]
    rows = lax.broadcasted_iota(jnp.int32, (_R, _C), 0)
    m = (rid == rows).astype(jnp.float32)
    counts = jnp.maximum(jnp.sum(m, axis=1, keepdims=True), 1.0)
    mn = m / counts
    pooled = jnp.dot(mn, feats_ref[:, :], preferred_element_type=jnp.float32)
    for b in range(_B):
        pb = pooled[:, b * _F:(b + 1) * _F]
        ob = jnp.dot(pb, wc_ref[:, :], preferred_element_type=jnp.float32)
        out_ref[b * _R:(b + 1) * _R, :] = ob + bc_ref[:, :]


def kernel(x, region_ids, kernels, biases, W, b):
    xr = x.reshape(_B * _C, _T)
    xp = jnp.pad(xr, ((0, 0), (0, _TP - _T))).astype(jnp.bfloat16)
    k2 = kernels.reshape(_K, _KL).astype(jnp.float32)

    thr = jnp.repeat(-biases.astype(jnp.float32), _L).reshape(1, _K * _L)
    thr = thr.astype(jnp.bfloat16)
    lane_t = jnp.arange(_K * _L) % _L
    maskv = (lane_t < (_TV - (_NCH - 1) * _L)).astype(jnp.bfloat16)
    maskv = maskv.reshape(1, _K * _L)
    tones = (jnp.arange(_K * _L)[:, None] // _L
             == jnp.arange(_K)[None, :]).astype(jnp.bfloat16)
    b2 = biases.reshape(1, _K).astype(jnp.float32)

    ncb = _C // _G
    feats_t = pl.pallas_call(
        _feats_kernel,
        grid=(_B * _C // _G,),
        in_specs=[
            pl.BlockSpec((_G, _TP), lambda i: (i, 0)),
            pl.BlockSpec((_K, _KL), lambda i: (0, 0)),
            pl.BlockSpec((1, _K * _L), lambda i: (0, 0)),
            pl.BlockSpec((1, _K * _L), lambda i: (0, 0)),
            pl.BlockSpec((_K * _L, _K), lambda i: (0, 0)),
            pl.BlockSpec((1, _K), lambda i: (0, 0)),
        ],
        out_specs=pl.BlockSpec((_G, _F), lambda i: (i % ncb, i // ncb)),
        out_shape=jax.ShapeDtypeStruct((_C, _B * _F), jnp.float32),
        scratch_shapes=[pltpu.VMEM((_W, _K * _L), jnp.bfloat16)],
    )(xp, k2, thr, maskv, tones, b2)

    feats_2d = feats_t
    wc = W.transpose(1, 0, 2).reshape(_F, _NB * _DO)
    bc = b.reshape(1, _NB * _DO)
    rid = region_ids.astype(jnp.int32).reshape(1, _C)

    out = pl.pallas_call(
        _pool_head_kernel,
        in_specs=[
            pl.BlockSpec((1, _C), lambda: (0, 0)),
            pl.BlockSpec((_C, _B * _F), lambda: (0, 0)),
            pl.BlockSpec((_F, _NB * _DO), lambda: (0, 0)),
            pl.BlockSpec((1, _NB * _DO), lambda: (0, 0)),
        ],
        out_specs=pl.BlockSpec((_B * _R, _NB * _DO), lambda: (0, 0)),
        out_shape=jax.ShapeDtypeStruct((_B * _R, _NB * _DO), jnp.float32),
    )(rid, feats_2d, wc, bc)

    out = out.reshape(_B, _R, _NB, _DO).transpose(0, 2, 1, 3)
    return out.reshape(_B, _NB, _R * _DO)


# speedup vs baseline: 77.9914x; 1.1579x over previous
import jax
import jax.numpy as jnp
from jax import lax
from jax.experimental import pallas as pl
from jax.experimental.pallas import tpu as pltpu

_B, _C, _T = 16, 64, 1000
_K = 64
_KL = 9
_TV = _T - _KL + 1
_R = 8
_NB = 4
_DO = 64
_F = 2 * _K

_L = 128
_NCH = 8
_W = _L + _KL - 1
_TP = _NCH * _L + (_KL - 1)
_G = 32


def _feats_kernel(x_ref, kern_ref, thr_ref, mask_ref, tones_ref,
                  out_ref, m_ref):
    @pl.when(pl.program_id(0) == 0)
    def _build_m():
        d = (lax.broadcasted_iota(jnp.int32, (_W, _L), 0)
             - lax.broadcasted_iota(jnp.int32, (_W, _L), 1))
        diags = [(d == j).astype(jnp.bfloat16) for j in range(_KL)]
        for k in range(_K):
            tb = diags[0] * kern_ref[k, 0].astype(jnp.bfloat16)
            for j in range(1, _KL):
                tb = tb + diags[j] * kern_ref[k, j].astype(jnp.bfloat16)
            m_ref[:_W, k * _L:(k + 1) * _L] = tb
        m_ref[_W:, :] = -thr_ref[:, :]

    a = jnp.concatenate(
        [x_ref[:, ch * _L:ch * _L + _W] for ch in range(_NCH)], axis=0)
    ones_col = jnp.ones((_NCH * _G, 1), jnp.bfloat16)
    a = jnp.concatenate([a, ones_col], axis=1)
    conv = jnp.dot(a, m_ref[:, :],
                   preferred_element_type=jnp.float32).astype(jnp.bfloat16)
    nrows = (_NCH - 1) * _G
    conv_main = conv[:nrows, :]
    conv_last = conv[nrows:, :]
    big = jnp.bfloat16(3e38)
    one = jnp.bfloat16(1.0)
    zero = jnp.bfloat16(0.0)
    ind_main = jnp.minimum(jnp.maximum(conv_main, zero) * big, one)
    ind_last = (jnp.minimum(jnp.maximum(conv_last, zero) * big, one)
                * mask_ref[:, :])
    ind = jnp.concatenate([ind_main, ind_last], axis=0)
    counts = jnp.dot(ind, tones_ref[:, :], preferred_element_type=jnp.float32)
    total = counts[:_G, :]
    for ch in range(1, _NCH):
        total = total + counts[ch * _G:(ch + 1) * _G, :]
    ppv = total * (1.0 / _TV)
    mb = mask_ref[:, :] > zero
    neg = jnp.full(conv_last.shape, jnp.bfloat16(-3e38))
    conv_last_m = jnp.where(mb, conv_last, neg)
    m = conv_main[:_G, :]
    for ch in range(1, _NCH - 1):
        m = jnp.maximum(m, conv_main[ch * _G:(ch + 1) * _G, :])
    m = jnp.maximum(m, conv_last_m)
    m3 = m.reshape(_G, _K, _L)
    mx = jnp.max(m3, axis=-1).astype(jnp.float32)
    out_ref[:, :] = jnp.concatenate([ppv, mx], axis=1)


def _pool_head_kernel(rid_ref, feats_ref, wc_ref, bc_ref, out_ref):
    rid = rid_ref[:, :]
    rows = lax.broadcasted_iota(jnp.int32, (_R, _C), 0)
    m = (rid == rows).astype(jnp.float32)
    counts = jnp.maximum(jnp.sum(m, axis=1, keepdims=True), 1.0)
    mn = m / counts
    pooled = jnp.dot(mn, feats_ref[:, :], preferred_element_type=jnp.float32)
    for b in range(_B):
        pb = pooled[:, b * _F:(b + 1) * _F]
        ob = jnp.dot(pb, wc_ref[:, :], preferred_element_type=jnp.float32)
        out_ref[b * _R:(b + 1) * _R, :] = ob + bc_ref[:, :]


def kernel(x, region_ids, kernels, biases, W, b):
    xr = x.reshape(_B * _C, _T)
    xp = jnp.pad(xr, ((0, 0), (0, _TP - _T))).astype(jnp.bfloat16)
    k2 = kernels.reshape(_K, _KL).astype(jnp.float32)

    thr = jnp.repeat(-biases.astype(jnp.float32), _L).reshape(1, _K * _L)
    thr = thr.astype(jnp.bfloat16)
    lane_t = jnp.arange(_K * _L) % _L
    maskv = (lane_t < (_TV - (_NCH - 1) * _L)).astype(jnp.bfloat16)
    maskv = maskv.reshape(1, _K * _L)
    tones = (jnp.arange(_K * _L)[:, None] // _L
             == jnp.arange(_K)[None, :]).astype(jnp.bfloat16)

    ncb = _C // _G
    feats_2d = pl.pallas_call(
        _feats_kernel,
        grid=(_B * _C // _G,),
        in_specs=[
            pl.BlockSpec((_G, _TP), lambda i: (i, 0)),
            pl.BlockSpec((_K, _KL), lambda i: (0, 0)),
            pl.BlockSpec((1, _K * _L), lambda i: (0, 0)),
            pl.BlockSpec((1, _K * _L), lambda i: (0, 0)),
            pl.BlockSpec((_K * _L, _K), lambda i: (0, 0)),
        ],
        out_specs=pl.BlockSpec((_G, _F), lambda i: (i % ncb, i // ncb)),
        out_shape=jax.ShapeDtypeStruct((_C, _B * _F), jnp.float32),
        scratch_shapes=[pltpu.VMEM((_W + 1, _K * _L), jnp.bfloat16)],
    )(xp, k2, thr, maskv, tones)

    wc = W.transpose(1, 0, 2).reshape(_F, _NB * _DO)
    bc = b.reshape(1, _NB * _DO)
    rid = region_ids.astype(jnp.int32).reshape(1, _C)

    out = pl.pallas_call(
        _pool_head_kernel,
        in_specs=[
            pl.BlockSpec((1, _C), lambda: (0, 0)),
            pl.BlockSpec((_C, _B * _F), lambda: (0, 0)),
            pl.BlockSpec((_F, _NB * _DO), lambda: (0, 0)),
            pl.BlockSpec((1, _NB * _DO), lambda: (0, 0)),
        ],
        out_specs=pl.BlockSpec((_B * _R, _NB * _DO), lambda: (0, 0)),
        out_shape=jax.ShapeDtypeStruct((_B * _R, _NB * _DO), jnp.float32),
    )(rid, feats_2d, wc, bc)

    out = out.reshape(_B, _R, _NB, _DO).transpose(0, 2, 1, 3)
    return out.reshape(_B, _NB, _R * _DO)
